# baseline (speedup 1.0000x reference)
import functools

import jax
import jax.numpy as jnp
from jax import lax
from jax.experimental import pallas as pl
from jax.experimental.pallas import tpu as pltpu
from jax.experimental.pallas import tpu_sc as plsc

_T = 64
_D = 784
_K = 200
_TT = 8

_ONE_BITS = 0x3F800000

_NV = _D // 16

_NC, _NS, _L = 2, 16, 16
_NW = _NC * _NS
_RPW = _T // _NW


def _splat_sum_f(acc):
    total = acc[0]
    for i in range(1, 16):
        total = total + acc[i]
    return lax.broadcast_in_dim(total, (16,), ())


def _sc_body(mask_hbm, attn_hbm, m_v, o_v):
    wid = lax.axis_index("s") * _NC + lax.axis_index("c")
    row0 = wid * _RPW
    pltpu.sync_copy(mask_hbm.at[pl.ds(row0, _RPW)], m_v)

    kvec = jnp.full((16,), float(_K), jnp.float32)
    zero_f = jnp.zeros((16,), jnp.float32)
    one_f = jnp.float32(1.0)
    nil_f = jnp.float32(0.0)
    lane = lax.iota(jnp.int32, 16)

    for r in range(_RPW):
        def sig_body(j, _):
            m = m_v[r, pl.ds(j * 16, 16)]
            o_v[r, pl.ds(j * 16, 16)] = 1.0 / (1.0 + jnp.exp(-m))
            return 0

        lax.fori_loop(0, _NV, sig_body, 0)

        def count_ge(thr_vec):
            def body(j, acc):
                hit = o_v[r, pl.ds(j * 16, 16)] >= thr_vec
                return acc + jnp.where(hit, one_f, nil_f)

            return _splat_sum_f(lax.fori_loop(0, _NV, body, zero_f))

        def bis(_, carry):
            lo, hi = carry
            mid = (lo + hi) * jnp.float32(0.5)
            take = count_ge(mid) >= kvec
            return jnp.where(take, mid, lo), jnp.where(take, hi, mid)

        lo, _ = lax.fori_loop(0, 40, bis,
                              (jnp.zeros((16,), jnp.float32),
                               jnp.ones((16,), jnp.float32)))

        def count_gt(thr_vec):
            def body(j, acc):
                hit = o_v[r, pl.ds(j * 16, 16)] > thr_vec
                return acc + jnp.where(hit, one_f, nil_f)

            return _splat_sum_f(lax.fori_loop(0, _NV, body, zero_f))

        need = kvec - count_gt(lo)

        def cnt_eq_le(cv):
            def body(j, acc):
                b = o_v[r, pl.ds(j * 16, 16)]
                col = lane + j * 16
                hit = (b == lo) & (col <= cv)
                return acc + jnp.where(hit, one_f, nil_f)

            return _splat_sum_f(lax.fori_loop(0, _NV, body, zero_f))

        def bis2(_, carry):
            lo2, hi2 = carry
            mid = lax.shift_right_logical(lo2 + hi2, 1)
            ok = cnt_eq_le(mid) >= need
            return jnp.where(ok, lo2, mid + 1), jnp.where(ok, mid, hi2)

        _, cstar = lax.fori_loop(0, 10, bis2,
                                 (jnp.zeros((16,), jnp.int32),
                                  jnp.full((16,), _D - 1, jnp.int32)))

        def wr_body(j, _):
            v = o_v[r, pl.ds(j * 16, 16)]
            col = lane + j * 16
            keep = (v > lo) | ((v == lo) & (col <= cstar))
            o_v[r, pl.ds(j * 16, 16)] = jnp.where(keep, v, nil_f)
            return 0

        lax.fori_loop(0, _NV, wr_body, 0)

    pltpu.sync_copy(o_v, attn_hbm.at[pl.ds(row0, _RPW)])


_sc_attention = functools.partial(
    pl.kernel,
    mesh=plsc.VectorSubcoreMesh(core_axis_name="c", subcore_axis_name="s"),
    out_type=jax.ShapeDtypeStruct((_T, _D), jnp.float32),
    scratch_types=[
        pltpu.VMEM((_RPW, _D), jnp.float32),
        pltpu.VMEM((_RPW, _D), jnp.float32),
    ],
)(_sc_body)


def _mul_body(attn_ref, xt_ref, out_ref):
    out_ref[...] = attn_ref[...][:, :, None] * xt_ref[...][None, :, :]


def kernel(x, attention_mask):
    x = x.reshape(-1, _D)
    b = x.shape[0]
    xt = jnp.swapaxes(x, 0, 1)

    attention = _sc_attention(attention_mask)

    out_tdb = pl.pallas_call(
        _mul_body,
        grid=(_T // _TT,),
        in_specs=[
            pl.BlockSpec((_TT, _D), lambda i: (i, 0)),
            pl.BlockSpec((_D, b), lambda i: (0, 0)),
        ],
        out_specs=pl.BlockSpec((_TT, _D, b), lambda i: (i, 0, 0)),
        out_shape=jax.ShapeDtypeStruct((_T, _D, b), jnp.float32),
        compiler_params=pltpu.CompilerParams(
            dimension_semantics=("arbitrary",),
        ),
    )(attention, xt)
    return_value = jnp.transpose(out_tdb, (2, 0, 1))
    return (return_value, attention)


# speedup vs baseline: 1.1168x; 1.1168x over previous
import functools

import jax
import jax.numpy as jnp
from jax import lax
from jax.experimental import pallas as pl
from jax.experimental.pallas import tpu as pltpu
from jax.experimental.pallas import tpu_sc as plsc

_T = 64
_D = 784
_K = 200
_TT = 8

_NV = _D // 16

_NC, _NS = 2, 16
_NW = _NC * _NS
_RPW = _T // _NW


def _splat_sum_f(acc):
    total = acc[0]
    for i in range(1, 16):
        total = total + acc[i]
    return lax.broadcast_in_dim(total, (16,), ())


def _sc_body(mask_hbm, attn_hbm, m_v, o_v):
    wid = lax.axis_index("s") * _NC + lax.axis_index("c")
    row0 = wid * _RPW
    pltpu.sync_copy(mask_hbm.at[pl.ds(row0, _RPW)], m_v)

    kvec = jnp.full((16,), float(_K), jnp.float32)
    zero_f = jnp.zeros((16,), jnp.float32)
    one_f = jnp.float32(1.0)
    nil_f = jnp.float32(0.0)
    lane = lax.iota(jnp.int32, 16)

    for r in range(_RPW):
        def sig_body(j, _):
            m = m_v[r, pl.ds(j * 16, 16)]
            o_v[r, pl.ds(j * 16, 16)] = 1.0 / (1.0 + jnp.exp(-m))
            return 0

        lax.fori_loop(0, _NV, sig_body, 0)

        def count_ge(thr_vec):
            def body(jj, acc):
                for u in range(7):
                    j = jj * 7 + u
                    hit = o_v[r, pl.ds(j * 16, 16)] >= thr_vec
                    acc = acc + jnp.where(hit, one_f, nil_f)
                return acc

            return _splat_sum_f(lax.fori_loop(0, 7, body, zero_f))

        def bis(_, carry):
            lo, hi = carry
            mid = (lo + hi) * jnp.float32(0.5)
            take = count_ge(mid) >= kvec
            return jnp.where(take, mid, lo), jnp.where(take, hi, mid)

        lo, _ = lax.fori_loop(0, 30, bis,
                              (jnp.full((16,), 0.25, jnp.float32),
                               jnp.full((16,), 0.75, jnp.float32)))

        def count_gt(thr_vec):
            def body(jj, acc):
                for u in range(7):
                    j = jj * 7 + u
                    hit = o_v[r, pl.ds(j * 16, 16)] > thr_vec
                    acc = acc + jnp.where(hit, one_f, nil_f)
                return acc

            return _splat_sum_f(lax.fori_loop(0, 7, body, zero_f))

        need = kvec - count_gt(lo)

        def cnt_eq_le(cv):
            def body(jj, acc):
                for u in range(7):
                    j = jj * 7 + u
                    b = o_v[r, pl.ds(j * 16, 16)]
                    col = lane + j * 16
                    hit = (b == lo) & (col <= cv)
                    acc = acc + jnp.where(hit, one_f, nil_f)
                return acc

            return _splat_sum_f(lax.fori_loop(0, 7, body, zero_f))

        def bis2(_, carry):
            lo2, hi2 = carry
            mid = lax.shift_right_logical(lo2 + hi2, 1)
            ok = cnt_eq_le(mid) >= need
            return jnp.where(ok, lo2, mid + 1), jnp.where(ok, mid, hi2)

        _, cstar = lax.fori_loop(0, 10, bis2,
                                 (jnp.zeros((16,), jnp.int32),
                                  jnp.full((16,), _D - 1, jnp.int32)))

        def wr_body(j, _):
            v = o_v[r, pl.ds(j * 16, 16)]
            col = lane + j * 16
            keep = (v > lo) | ((v == lo) & (col <= cstar))
            o_v[r, pl.ds(j * 16, 16)] = jnp.where(keep, v, nil_f)
            return 0

        lax.fori_loop(0, _NV, wr_body, 0)

    pltpu.sync_copy(o_v, attn_hbm.at[pl.ds(row0, _RPW)])


_sc_attention = functools.partial(
    pl.kernel,
    mesh=plsc.VectorSubcoreMesh(core_axis_name="c", subcore_axis_name="s"),
    out_type=jax.ShapeDtypeStruct((_T, _D), jnp.float32),
    scratch_types=[
        pltpu.VMEM((_RPW, _D), jnp.float32),
        pltpu.VMEM((_RPW, _D), jnp.float32),
    ],
)(_sc_body)


def _mul_body(attn_ref, xt_ref, out_ref):
    out_ref[...] = attn_ref[...][:, :, None] * xt_ref[...][None, :, :]


def kernel(x, attention_mask):
    x = x.reshape(-1, _D)
    b = x.shape[0]
    xt = jnp.swapaxes(x, 0, 1)

    attention = _sc_attention(attention_mask)

    out_tdb = pl.pallas_call(
        _mul_body,
        grid=(_T // _TT,),
        in_specs=[
            pl.BlockSpec((_TT, _D), lambda i: (i, 0)),
            pl.BlockSpec((_D, b), lambda i: (0, 0)),
        ],
        out_specs=pl.BlockSpec((_TT, _D, b), lambda i: (i, 0, 0)),
        out_shape=jax.ShapeDtypeStruct((_T, _D, b), jnp.float32),
        compiler_params=pltpu.CompilerParams(
            dimension_semantics=("arbitrary",),
        ),
    )(attention, xt)
    return_value = jnp.transpose(out_tdb, (2, 0, 1))
    return (return_value, attention)
